# baseline (speedup 1.0000x reference)
import jax
import jax.numpy as jnp
from jax.experimental import pallas as pl


def _copy2_body(src_ref, src2_ref, dst_ref, dst2_ref):
    dst_ref[...] = src_ref[...]
    dst2_ref[...] = src2_ref[...]


def kernel(output, bank):
    dim, size = bank.shape
    b, d = output.shape
    n2 = b * d
    out2 = output.reshape(dim, n2 // dim)
    blk = 16384
    grid = size // blk
    blk2 = (n2 // dim) // grid
    snap, out_copy = pl.pallas_call(
        _copy2_body,
        grid=(grid,),
        in_specs=[
            pl.BlockSpec((dim, blk), lambda i: (0, i)),
            pl.BlockSpec((dim, blk2), lambda i: (0, i)),
        ],
        out_specs=[
            pl.BlockSpec((dim, blk), lambda i: (0, i)),
            pl.BlockSpec((dim, blk2), lambda i: (0, i)),
        ],
        out_shape=[
            jax.ShapeDtypeStruct(bank.shape, bank.dtype),
            jax.ShapeDtypeStruct(out2.shape, output.dtype),
        ],
    )(bank, out2)
    return (out_copy.reshape(b, d), snap)


# speedup vs baseline: 1.2367x; 1.2367x over previous
import jax
import jax.numpy as jnp
from jax.experimental import pallas as pl


def _copy2_body(src_ref, src2_ref, dst_ref, dst2_ref):
    dst_ref[...] = src_ref[...]
    dst2_ref[...] = src2_ref[...]


def kernel(output, bank):
    dim, size = bank.shape
    b, d = output.shape
    blk = 16384
    grid = size // blk
    rblk = b // grid
    snap, out_copy = pl.pallas_call(
        _copy2_body,
        grid=(grid,),
        in_specs=[
            pl.BlockSpec((dim, blk), lambda i: (0, i)),
            pl.BlockSpec((rblk, d), lambda i: (i, 0)),
        ],
        out_specs=[
            pl.BlockSpec((dim, blk), lambda i: (0, i)),
            pl.BlockSpec((rblk, d), lambda i: (i, 0)),
        ],
        out_shape=[
            jax.ShapeDtypeStruct(bank.shape, bank.dtype),
            jax.ShapeDtypeStruct(output.shape, output.dtype),
        ],
    )(bank, output)
    return (out_copy, snap)
